# Initial kernel scaffold; baseline (speedup 1.0000x reference)
import jax
import jax.numpy as jnp
from jax.experimental import pallas as pl


def kernel(x, durations, val_ind):
    raise NotImplementedError("write your pallas kernel here")


# speedup vs baseline: 2.3561x; 2.3561x over previous
import functools

import jax
import jax.numpy as jnp
from jax import lax
from jax.experimental import pallas as pl
from jax.experimental.pallas import tpu as pltpu
from jax.experimental.pallas import tpu_sc as plsc

B, P, F, D = 16, 512, 2048, 512
NC, NS = 2, 16
NW = NC * NS
ROWS_PER_W = (B * F) // NW
CHUNK = 64
NCHUNK = ROWS_PER_W // CHUNK
FRAMES_PER_BATCH_TILE = F // (ROWS_PER_W)


@functools.partial(
    pl.kernel,
    out_type=jax.ShapeDtypeStruct((B * F, D), jnp.float32),
    mesh=plsc.VectorSubcoreMesh(core_axis_name="c", subcore_axis_name="s"),
    scratch_types=[
        pltpu.VMEM((NCHUNK, CHUNK), jnp.int32),
        pltpu.VMEM((CHUNK, D), jnp.float32),
        pltpu.VMEM((CHUNK, D), jnp.float32),
        pltpu.SemaphoreType.DMA,
        pltpu.SemaphoreType.DMA,
    ],
)
def _sc_gather(x_hbm, vi_hbm, out_hbm, idx_v, buf0, buf1, gsem, ssem):
    wid = lax.axis_index("s") * NC + lax.axis_index("c")
    base = wid * ROWS_PER_W

    pltpu.sync_copy(vi_hbm.at[wid], idx_v)

    off = (wid // (F // ROWS_PER_W)) * P
    for j in range(NCHUNK):
        for k in range(CHUNK // 16):
            sl = (j, pl.ds(k * 16, 16))
            idx_v[sl] = idx_v[sl] + off

    bufs = (buf0, buf1)

    def start_gather(j):
        return pltpu.async_copy(x_hbm.at[idx_v.at[j]], bufs[j % 2], gsem)

    def start_scatter(j):
        dst = out_hbm.at[pl.ds(base + j * CHUNK, CHUNK)]
        return pltpu.async_copy(bufs[j % 2], dst, ssem)

    gathers = [None] * NCHUNK
    scatters = [None] * NCHUNK
    gathers[0] = start_gather(0)
    for j in range(NCHUNK):
        if j + 1 < NCHUNK:
            if j >= 1:
                scatters[j - 1].wait()
            gathers[j + 1] = start_gather(j + 1)
        gathers[j].wait()
        scatters[j] = start_scatter(j)
    scatters[NCHUNK - 2].wait()
    scatters[NCHUNK - 1].wait()


def _mask_body(vi_ref, o_ref):
    o_ref[...] = vi_ref[...] != (P - 1)


def kernel(x, durations, val_ind):
    del durations
    table = x.reshape(B * P, D)
    vi = val_ind.reshape(NW, NCHUNK, CHUNK)
    out = _sc_gather(table, vi)
    tgt_mask = pl.pallas_call(
        _mask_body,
        out_shape=jax.ShapeDtypeStruct((B, F), jnp.bool_),
    )(val_ind)
    return out.reshape(B, F, D), tgt_mask[..., None]
